# Initial kernel scaffold; baseline (speedup 1.0000x reference)
import jax
import jax.numpy as jnp
from jax.experimental import pallas as pl


def kernel(hidden_states, W):
    raise NotImplementedError("write your pallas kernel here")


# speedup vs baseline: 3.4174x; 3.4174x over previous
import functools

import jax
import jax.numpy as jnp
from jax.experimental import pallas as pl

B, S, H, E = 4, 2048, 4096, 64
N = B * S
K = 160

TILE = 512
GRID_A = N // TILE


def _router_body(x_ref, w_ref, probs_ref):
    x = x_ref[...]
    w = w_ref[...]
    logits = jax.lax.dot_general(
        x, w, (((1,), (1,)), ((), ())),
        preferred_element_type=jnp.float32,
        precision=jax.lax.Precision.DEFAULT,
    )
    m = jnp.max(logits, axis=1, keepdims=True)
    e = jnp.exp(logits - m)
    probs_ref[...] = e / jnp.sum(e, axis=1, keepdims=True)


def _select_body(probs_ref, disp_ref, comb_ref):
    p = probs_ref[...]
    bits = jax.lax.bitcast_convert_type(p, jnp.int32)

    def val_step(i, lo):
        cand = lo | (1 << (30 - i))
        cnt = jnp.sum((bits >= cand).astype(jnp.int32), axis=0, keepdims=True)
        return jnp.where(cnt >= K, cand, lo)

    t = jax.lax.fori_loop(0, 31, val_step, jnp.zeros((1, E), jnp.int32))

    gt = jnp.sum((bits > t).astype(jnp.int32), axis=0, keepdims=True)
    needed = K - gt
    ties = bits == t
    idx = jax.lax.broadcasted_iota(jnp.int32, (N, E), 0)

    def idx_step(i, cut):
        cand = cut | (1 << (12 - i))
        cnt = jnp.sum((ties & (idx < cand)).astype(jnp.int32), axis=0,
                      keepdims=True)
        return jnp.where(cnt < needed, cand, cut)

    cut = jax.lax.fori_loop(0, 13, idx_step, jnp.zeros((1, E), jnp.int32))

    sel = (bits > t) | (ties & (idx <= cut))
    disp = jnp.where(sel, p, 0.0)
    dsum = jnp.sum(disp, axis=1, keepdims=True)
    comb = jnp.where(dsum > 0, disp / dsum, 0.0)
    disp_ref[...] = disp
    comb_ref[...] = comb


@functools.partial(jax.jit, static_argnames=("interpret",))
def kernel(hidden_states, W, interpret=False):
    x = hidden_states.reshape(N, H)
    probs = pl.pallas_call(
        _router_body,
        grid=(GRID_A,),
        in_specs=[
            pl.BlockSpec((TILE, H), lambda i: (i, 0)),
            pl.BlockSpec((E, H), lambda i: (0, 0)),
        ],
        out_specs=pl.BlockSpec((TILE, E), lambda i: (i, 0)),
        out_shape=jax.ShapeDtypeStruct((N, E), jnp.float32),
        interpret=interpret,
    )(x, W)

    disp, comb = pl.pallas_call(
        _select_body,
        out_shape=[
            jax.ShapeDtypeStruct((N, E), jnp.float32),
            jax.ShapeDtypeStruct((N, E), jnp.float32),
        ],
        interpret=interpret,
    )(probs)

    shape = (B, S, E)
    return (disp.reshape(shape), comb.reshape(shape),
            jnp.array(0.0, dtype=jnp.float32), probs.reshape(shape))
